# baseline (speedup 1.0000x reference)
import functools

import jax
import jax.numpy as jnp
from jax import lax
from jax.experimental import pallas as pl
from jax.experimental.pallas import tpu as pltpu
from jax.experimental.pallas import tpu_sc as plsc

N = 10000
NP = 10240
SLAB = 640
NNZ = 160000
NBLK = 79
BLK = 128
PADROW = 10200
W = 64

_GDN = lax.GatherDimensionNumbers(
    offset_dims=(), collapsed_slice_dims=(0,), start_index_map=(0,))


SLABC = 128
NCHUNK = SLAB // SLABC


def _sc_body(z0_hbm, z1_hbm, nidx_hbm, eidx_hbm, u0_hbm, u1_hbm, d_hbm,
             bufS, bufT, Dn, Be,
             nidx_v, eidx_v, rows_v, rows2_v, slab_v, binv_v, dinv_v,
             dvec_v, ones_v, gsem0, gsem1, dsem):
    c = lax.axis_index("c")
    s = lax.axis_index("s")
    r0 = s * SLAB

    zvec = jnp.zeros((16,), jnp.float32)
    onevec = jnp.ones((16,), jnp.float32)

    def fill_slab_zeros():
        def fz(i, _):
            for c4 in range(4):
                slab_v[i, pl.ds(c4 * 16, 16)] = zvec
            return 0
        lax.fori_loop(0, SLABC, fz, 0)

    for j in range(BLK // 16):
        ones_v[pl.ds(j * 16, 16)] = onevec
    for j in range(SLAB // 16):
        binv_v[pl.ds(j * 16, 16)] = zvec

    pltpu.sync_copy(nidx_hbm.at[s], nidx_v)
    pltpu.sync_copy(eidx_hbm.at[s], eidx_v)
    for k in range(NCHUNK):
        ck = pl.ds(r0 + k * SLABC, SLABC)

        @pl.when(c == 0)
        def _():
            pltpu.sync_copy(z0_hbm.at[ck], slab_v)

        @pl.when(c == 1)
        def _():
            pltpu.sync_copy(z1_hbm.at[ck], slab_v)

        pltpu.sync_copy(slab_v, bufS.at[ck])
    fill_slab_zeros()
    for k in range(NCHUNK):
        pltpu.sync_copy(slab_v, bufT.at[pl.ds(r0 + k * SLABC, SLABC)])
    pltpu.sync_copy(binv_v, Dn.at[pl.ds(r0, SLAB)])
    pltpu.sync_copy(binv_v, Be.at[pl.ds(r0, SLAB)])
    plsc.subcore_barrier()

    def deg_body(j, _):
        pltpu.async_copy(ones_v, Dn.at[nidx_v.at[j]], dsem, add=True)
        pltpu.async_copy(ones_v, Be.at[eidx_v.at[j]], dsem, add=True)
        return 0
    lax.fori_loop(0, NBLK, deg_body, 0)

    def deg_drain(j, _):
        pltpu.make_async_copy(ones_v, Dn.at[nidx_v.at[0]], dsem).wait()
        pltpu.make_async_copy(ones_v, Be.at[eidx_v.at[0]], dsem).wait()
        return 0
    lax.fori_loop(0, NBLK, deg_drain, 0)
    plsc.subcore_barrier()

    pltpu.sync_copy(Be.at[pl.ds(r0, SLAB)], binv_v)
    pltpu.sync_copy(Dn.at[pl.ds(r0, SLAB)], dinv_v)

    def inv_body(i, _):
        be = binv_v[pl.ds(i * 16, 16)]
        binv_v[pl.ds(i * 16, 16)] = jnp.where(be > 0, 1.0 / be, 0.0)
        dn = dinv_v[pl.ds(i * 16, 16)]
        dinv_v[pl.ds(i * 16, 16)] = jnp.where(dn > 0, 1.0 / dn, 0.0)
        dvec_v[pl.ds(i * 16, 16)] = jnp.where(dn > 0, 1.0, 0.0)
        return 0
    lax.fori_loop(0, SLAB // 16, inv_body, 0)

    @pl.when(c == 0)
    def _():
        pltpu.sync_copy(dvec_v, d_hbm.at[pl.ds(r0, SLAB)])

    def pass_fn(src, dst, sidx, didx):
        pltpu.async_copy(src.at[sidx.at[0]], rows_v, gsem0)
        pltpu.async_copy(src.at[sidx.at[1]], rows2_v, gsem1)

        def pair(i, _):
            j = i * 2
            pltpu.make_async_copy(src.at[sidx.at[j]], rows_v, gsem0).wait()
            pltpu.sync_copy(rows_v, dst.at[didx.at[j]], add=True)

            @pl.when(j + 2 < NBLK)
            def _():
                pltpu.async_copy(src.at[sidx.at[j + 2]], rows_v, gsem0)

            pltpu.make_async_copy(
                src.at[sidx.at[j + 1]], rows2_v, gsem1).wait()
            pltpu.sync_copy(rows2_v, dst.at[didx.at[j + 1]], add=True)

            @pl.when(j + 3 < NBLK)
            def _():
                pltpu.async_copy(src.at[sidx.at[j + 3]], rows2_v, gsem1)
            return 0
        lax.fori_loop(0, NBLK // 2, pair, 0)
        jt = NBLK - 1
        pltpu.make_async_copy(src.at[sidx.at[jt]], rows_v, gsem0).wait()
        pltpu.sync_copy(rows_v, dst.at[didx.at[jt]], add=True)
        plsc.subcore_barrier()

    def scale_chunk(scalevec, k):
        def sgroup(g, _):
            chunk = scalevec[pl.ds(k * SLABC + g * 16, 16)]
            for i in range(16):
                sv = lax.gather(
                    chunk, jnp.full((16, 1), i, jnp.int32), _GDN, (1,),
                    mode=lax.GatherScatterMode.PROMISE_IN_BOUNDS)
                r = g * 16 + i
                for c4 in range(4):
                    slab_v[r, pl.ds(c4 * 16, 16)] = (
                        slab_v[r, pl.ds(c4 * 16, 16)] * sv)
            return 0
        lax.fori_loop(0, SLABC // 16, sgroup, 0)

    def scale_zero(buf, scalevec, other):
        for k in range(NCHUNK):
            ck = pl.ds(r0 + k * SLABC, SLABC)
            pltpu.sync_copy(buf.at[ck], slab_v)
            scale_chunk(scalevec, k)
            pltpu.sync_copy(slab_v, buf.at[ck])
        fill_slab_zeros()
        for k in range(NCHUNK):
            pltpu.sync_copy(slab_v, other.at[pl.ds(r0 + k * SLABC, SLABC)])
        plsc.subcore_barrier()

    pass_fn(bufS, bufT, nidx_v, eidx_v)
    scale_zero(bufT, binv_v, bufS)
    pass_fn(bufT, bufS, eidx_v, nidx_v)
    scale_zero(bufS, dinv_v, bufT)
    pass_fn(bufS, bufT, nidx_v, eidx_v)
    scale_zero(bufT, binv_v, bufS)
    pass_fn(bufT, bufS, eidx_v, nidx_v)

    for k in range(NCHUNK):
        ck = pl.ds(r0 + k * SLABC, SLABC)
        pltpu.sync_copy(bufS.at[ck], slab_v)
        scale_chunk(dinv_v, k)

        @pl.when(c == 0)
        def _():
            pltpu.sync_copy(slab_v, u0_hbm.at[ck])

        @pl.when(c == 1)
        def _():
            pltpu.sync_copy(slab_v, u1_hbm.at[ck])


_sc_prop = functools.partial(
    pl.kernel,
    out_type=[jax.ShapeDtypeStruct((NP, W), jnp.float32),
              jax.ShapeDtypeStruct((NP, W), jnp.float32),
              jax.ShapeDtypeStruct((NP,), jnp.float32)],
    mesh=plsc.VectorSubcoreMesh(core_axis_name="c", subcore_axis_name="s"),
    compiler_params=pltpu.CompilerParams(use_tc_tiling_on_sc=False),
    scratch_types=[
        pltpu.VMEM_SHARED((NP, W), jnp.float32),
        pltpu.VMEM_SHARED((NP, W), jnp.float32),
        pltpu.VMEM_SHARED((NP,), jnp.float32),
        pltpu.VMEM_SHARED((NP,), jnp.float32),
        pltpu.VMEM((NBLK, BLK), jnp.int32),
        pltpu.VMEM((NBLK, BLK), jnp.int32),
        pltpu.VMEM((BLK, W), jnp.float32),
        pltpu.VMEM((BLK, W), jnp.float32),
        pltpu.VMEM((SLABC, W), jnp.float32),
        pltpu.VMEM((SLAB,), jnp.float32),
        pltpu.VMEM((SLAB,), jnp.float32),
        pltpu.VMEM((SLAB,), jnp.float32),
        pltpu.VMEM((BLK,), jnp.float32),
        pltpu.SemaphoreType.DMA,
        pltpu.SemaphoreType.DMA,
        pltpu.SemaphoreType.DMA,
    ],
)(_sc_body)


def _layer_norm(v, g, b):
    on = jnp.full((v.shape[-1], 1), 1.0 / v.shape[-1], jnp.float32)
    mu = lax.dot_general(v, on, (((1,), (0,)), ((), ())),
                         preferred_element_type=jnp.float32)
    m2 = lax.dot_general(v * v, on, (((1,), (0,)), ((), ())),
                         preferred_element_type=jnp.float32)
    var = m2 - mu * mu
    return (v - mu) * lax.rsqrt(var + 1e-5) * g + b


def _pre_body(x_ref, lng_ref, lnb_ref, w1_ref, b1_ref, g1_ref, bb1_ref,
              wc1_ref, z0_ref, z1_ref):
    zs = []
    for i in range(8):
        y = jax.nn.relu(_layer_norm(x_ref[i], lng_ref[...], lnb_ref[...]))
        y = lax.dot_general(y, w1_ref[...], (((1,), (1,)), ((), ())),
                            preferred_element_type=jnp.float32) + b1_ref[...]
        y = jax.nn.relu(_layer_norm(y, g1_ref[...], bb1_ref[...]))
        zs.append(lax.dot_general(y, wc1_ref[...], (((1,), (1,)), ((), ())),
                                  preferred_element_type=jnp.float32))
    z0_ref[...] = jnp.concatenate(zs[:4], axis=1)
    z1_ref[...] = jnp.concatenate(zs[4:], axis=1)


def _post_body(u0_ref, u1_ref, x_ref, d_ref, wc2_ref, bc1_ref, bc2_ref,
               g2_ref, bb2_ref, w2_ref, b2_ref, o_ref):
    wb = jnp.sum(wc2_ref[...] * bc1_ref[...][None, :], axis=1)
    u0 = u0_ref[...]
    u1 = u1_ref[...]
    db = d_ref[...]
    for i in range(8):
        ui = (u0 if i < 4 else u1)[:, (i % 4) * 16:(i % 4) * 16 + 16]
        c2 = lax.dot_general(ui, wc2_ref[...], (((1,), (1,)), ((), ())),
                             preferred_element_type=jnp.float32)
        c2 = c2 + db * wb[None, :] + bc2_ref[...]
        t = jax.nn.relu(_layer_norm(c2, g2_ref[...], bb2_ref[...]))
        y = lax.dot_general(t, w2_ref[...], (((1,), (1,)), ((), ())),
                            preferred_element_type=jnp.float32) + b2_ref[...]
        o_ref[i] = x_ref[i] + y


def _rep(shape):
    return pl.BlockSpec(shape, lambda nb: (0,) * len(shape))


def kernel(x, incident_matrix, ln_pre_g, ln_pre_b, lin1_W, lin1_b, ln1_g,
           ln1_b, conv1_W, conv1_b, conv2_W, conv2_b, ln2_g, ln2_b, lin2_W,
           lin2_b):
    B, n, C = x.shape
    R = 1024
    grid = (NP // R,)

    Z0, Z1 = pl.pallas_call(
        _pre_body,
        grid=grid,
        in_specs=[
            pl.BlockSpec((B, R, C), lambda nb: (0, nb, 0)),
            _rep((C,)), _rep((C,)),
            _rep((32, C)), _rep((32,)), _rep((32,)), _rep((32,)),
            _rep((16, 32)),
        ],
        out_specs=[pl.BlockSpec((R, W), lambda nb: (nb, 0)),
                   pl.BlockSpec((R, W), lambda nb: (nb, 0))],
        out_shape=[jax.ShapeDtypeStruct((NP, W), jnp.float32),
                   jax.ShapeDtypeStruct((NP, W), jnp.float32)],
    )(x, ln_pre_g, ln_pre_b, lin1_W, lin1_b, ln1_g, ln1_b, conv1_W)

    inc = incident_matrix.astype(jnp.int32).reshape(2, 16, 10000)
    pad = jnp.full((2, 16, NBLK * BLK - 10000), PADROW, jnp.int32)
    idx = jnp.concatenate([inc, pad], axis=2).reshape(2, 16, NBLK, BLK)
    U0, U1, d = _sc_prop(Z0, Z1, idx[0], idx[1])

    out = pl.pallas_call(
        _post_body,
        grid=grid,
        in_specs=[
            pl.BlockSpec((R, W), lambda nb: (nb, 0)),
            pl.BlockSpec((R, W), lambda nb: (nb, 0)),
            pl.BlockSpec((B, R, C), lambda nb: (0, nb, 0)),
            pl.BlockSpec((R, 1), lambda nb: (nb, 0)),
            _rep((64, 16)), _rep((16,)), _rep((64,)),
            _rep((64,)), _rep((64,)),
            _rep((C, 64)), _rep((C,)),
        ],
        out_specs=pl.BlockSpec((B, R, C), lambda nb: (0, nb, 0)),
        out_shape=jax.ShapeDtypeStruct((B, n, C), jnp.float32),
    )(U0, U1, x, d[:, None], conv2_W, conv1_b, conv2_b, ln2_g, ln2_b,
      lin2_W, lin2_b)
    return out


# speedup vs baseline: 1.1221x; 1.1221x over previous
import functools

import jax
import jax.numpy as jnp
from jax import lax
from jax.experimental import pallas as pl
from jax.experimental.pallas import tpu as pltpu
from jax.experimental.pallas import tpu_sc as plsc

N = 10000
NP = 10240
SLAB = 640
NNZ = 160000
NBLK = 125
BLK = 80
W = 64

_GDN = lax.GatherDimensionNumbers(
    offset_dims=(), collapsed_slice_dims=(0,), start_index_map=(0,))


SLABC = 128
NCHUNK = SLAB // SLABC


def _sc_body(z0_hbm, z1_hbm, nidx_hbm, eidx_hbm, u0_hbm, u1_hbm, d_hbm,
             bufS, bufT, Dn, Be,
             nidx_v, eidx_v, rows_v, rows2_v, rows3_v, slab_v, binv_v,
             dinv_v, dvec_v, ones_v, gsem0, gsem1, gsem2,
             ssem0, ssem1, ssem2, dsem):
    c = lax.axis_index("c")
    s = lax.axis_index("s")
    r0 = s * SLAB

    zvec = jnp.zeros((16,), jnp.float32)
    onevec = jnp.ones((16,), jnp.float32)

    def fill_slab_zeros():
        def fz(i, _):
            for c4 in range(4):
                slab_v[i, pl.ds(c4 * 16, 16)] = zvec
            return 0
        lax.fori_loop(0, SLABC, fz, 0)

    for j in range(BLK // 16):
        ones_v[pl.ds(j * 16, 16)] = onevec
    for j in range(SLAB // 16):
        binv_v[pl.ds(j * 16, 16)] = zvec

    pltpu.sync_copy(binv_v, Dn.at[pl.ds(r0, SLAB)])
    pltpu.sync_copy(binv_v, Be.at[pl.ds(r0, SLAB)])
    plsc.subcore_barrier()

    pltpu.sync_copy(nidx_hbm.at[s], nidx_v)
    pltpu.sync_copy(eidx_hbm.at[s], eidx_v)

    def deg_body(j, _):
        pltpu.async_copy(ones_v, Dn.at[nidx_v.at[j]], dsem, add=True)
        pltpu.async_copy(ones_v, Be.at[eidx_v.at[j]], dsem, add=True)
        return 0
    lax.fori_loop(0, NBLK, deg_body, 0)

    for k in range(NCHUNK):
        ck = pl.ds(r0 + k * SLABC, SLABC)

        @pl.when(c == 0)
        def _():
            pltpu.sync_copy(z0_hbm.at[ck], slab_v)

        @pl.when(c == 1)
        def _():
            pltpu.sync_copy(z1_hbm.at[ck], slab_v)

        pltpu.sync_copy(slab_v, bufS.at[ck])
    fill_slab_zeros()
    for k in range(NCHUNK):
        pltpu.sync_copy(slab_v, bufT.at[pl.ds(r0 + k * SLABC, SLABC)])

    def deg_drain(j, _):
        pltpu.make_async_copy(ones_v, Dn.at[nidx_v.at[0]], dsem).wait()
        pltpu.make_async_copy(ones_v, Be.at[eidx_v.at[0]], dsem).wait()
        return 0
    lax.fori_loop(0, NBLK, deg_drain, 0)
    plsc.subcore_barrier()

    pltpu.sync_copy(Be.at[pl.ds(r0, SLAB)], binv_v)
    pltpu.sync_copy(Dn.at[pl.ds(r0, SLAB)], dinv_v)

    def inv_body(i, _):
        be = binv_v[pl.ds(i * 16, 16)]
        binv_v[pl.ds(i * 16, 16)] = jnp.where(be > 0, 1.0 / be, 0.0)
        dn = dinv_v[pl.ds(i * 16, 16)]
        dinv_v[pl.ds(i * 16, 16)] = jnp.where(dn > 0, 1.0 / dn, 0.0)
        dvec_v[pl.ds(i * 16, 16)] = jnp.where(dn > 0, 1.0, 0.0)
        return 0
    lax.fori_loop(0, SLAB // 16, inv_body, 0)

    @pl.when(c == 0)
    def _():
        pltpu.sync_copy(dvec_v, d_hbm.at[pl.ds(r0, SLAB)])

    def pass_fn(src, dst, sidx, didx):
        rbufs = (rows_v, rows2_v, rows3_v)
        gsems = (gsem0, gsem1, gsem2)
        ssems = (ssem0, ssem1, ssem2)
        pltpu.async_copy(src.at[sidx.at[0]], rbufs[0], gsems[0])
        pltpu.async_copy(src.at[sidx.at[1]], rbufs[1], gsems[1])

        def section(j, b, prefetch):
            pltpu.make_async_copy(src.at[sidx.at[j]], rbufs[b],
                                  gsems[b]).wait()
            pltpu.async_copy(rbufs[b], dst.at[didx.at[j]], ssems[b],
                             add=True)
            if prefetch:
                b2 = (b + 2) % 3

                @pl.when(j >= 1)
                def _():
                    pltpu.make_async_copy(rbufs[b2], dst.at[didx.at[0]],
                                          ssems[b2]).wait()
                pltpu.async_copy(src.at[sidx.at[j + 2]], rbufs[b2],
                                 gsems[b2])

        def super3(i, _):
            for b in range(3):
                section(i * 3 + b, b, True)
            return 0
        lax.fori_loop(0, NBLK // 3, super3, 0)
        for j in range(NBLK - NBLK % 3, NBLK):
            section(j, j % 3, False)
        for j in range(NBLK - 3, NBLK):
            pltpu.make_async_copy(rbufs[j % 3], dst.at[didx.at[0]],
                                  ssems[j % 3]).wait()
        plsc.subcore_barrier()

    def scale_chunk(scalevec, k):
        def sgroup(g, _):
            chunk = scalevec[pl.ds(k * SLABC + g * 16, 16)]
            for i in range(16):
                sv = lax.gather(
                    chunk, jnp.full((16, 1), i, jnp.int32), _GDN, (1,),
                    mode=lax.GatherScatterMode.PROMISE_IN_BOUNDS)
                r = g * 16 + i
                for c4 in range(4):
                    slab_v[r, pl.ds(c4 * 16, 16)] = (
                        slab_v[r, pl.ds(c4 * 16, 16)] * sv)
            return 0
        lax.fori_loop(0, SLABC // 16, sgroup, 0)

    def scale_zero(buf, scalevec, other):
        for k in range(NCHUNK):
            ck = pl.ds(r0 + k * SLABC, SLABC)
            pltpu.sync_copy(buf.at[ck], slab_v)
            scale_chunk(scalevec, k)
            pltpu.sync_copy(slab_v, buf.at[ck])
        fill_slab_zeros()
        for k in range(NCHUNK):
            pltpu.sync_copy(slab_v, other.at[pl.ds(r0 + k * SLABC, SLABC)])
        plsc.subcore_barrier()

    pass_fn(bufS, bufT, nidx_v, eidx_v)
    scale_zero(bufT, binv_v, bufS)
    pass_fn(bufT, bufS, eidx_v, nidx_v)
    scale_zero(bufS, dinv_v, bufT)
    pass_fn(bufS, bufT, nidx_v, eidx_v)
    scale_zero(bufT, binv_v, bufS)
    pass_fn(bufT, bufS, eidx_v, nidx_v)

    for k in range(NCHUNK):
        ck = pl.ds(r0 + k * SLABC, SLABC)
        pltpu.sync_copy(bufS.at[ck], slab_v)
        scale_chunk(dinv_v, k)

        @pl.when(c == 0)
        def _():
            pltpu.sync_copy(slab_v, u0_hbm.at[ck])

        @pl.when(c == 1)
        def _():
            pltpu.sync_copy(slab_v, u1_hbm.at[ck])


_sc_prop = functools.partial(
    pl.kernel,
    out_type=[jax.ShapeDtypeStruct((NP, W), jnp.float32),
              jax.ShapeDtypeStruct((NP, W), jnp.float32),
              jax.ShapeDtypeStruct((NP,), jnp.float32)],
    mesh=plsc.VectorSubcoreMesh(core_axis_name="c", subcore_axis_name="s"),
    compiler_params=pltpu.CompilerParams(use_tc_tiling_on_sc=False),
    scratch_types=[
        pltpu.VMEM_SHARED((NP, W), jnp.float32),
        pltpu.VMEM_SHARED((NP, W), jnp.float32),
        pltpu.VMEM_SHARED((NP,), jnp.float32),
        pltpu.VMEM_SHARED((NP,), jnp.float32),
        pltpu.VMEM((NBLK, BLK), jnp.int32),
        pltpu.VMEM((NBLK, BLK), jnp.int32),
        pltpu.VMEM((BLK, W), jnp.float32),
        pltpu.VMEM((BLK, W), jnp.float32),
        pltpu.VMEM((BLK, W), jnp.float32),
        pltpu.VMEM((SLABC, W), jnp.float32),
        pltpu.VMEM((SLAB,), jnp.float32),
        pltpu.VMEM((SLAB,), jnp.float32),
        pltpu.VMEM((SLAB,), jnp.float32),
        pltpu.VMEM((BLK,), jnp.float32),
        pltpu.SemaphoreType.DMA,
        pltpu.SemaphoreType.DMA,
        pltpu.SemaphoreType.DMA,
        pltpu.SemaphoreType.DMA,
        pltpu.SemaphoreType.DMA,
        pltpu.SemaphoreType.DMA,
        pltpu.SemaphoreType.DMA,
    ],
)(_sc_body)


def _layer_norm(v, g, b):
    on = jnp.full((v.shape[-1], 1), 1.0 / v.shape[-1], jnp.float32)
    mu = lax.dot_general(v, on, (((1,), (0,)), ((), ())),
                         preferred_element_type=jnp.float32)
    m2 = lax.dot_general(v * v, on, (((1,), (0,)), ((), ())),
                         preferred_element_type=jnp.float32)
    var = m2 - mu * mu
    return (v - mu) * lax.rsqrt(var + 1e-5) * g + b


def _pre_body(x_ref, lng_ref, lnb_ref, w1_ref, b1_ref, g1_ref, bb1_ref,
              wc1_ref, z0_ref, z1_ref):
    zs = []
    for i in range(8):
        y = jax.nn.relu(_layer_norm(x_ref[i], lng_ref[...], lnb_ref[...]))
        y = lax.dot_general(y, w1_ref[...], (((1,), (1,)), ((), ())),
                            preferred_element_type=jnp.float32) + b1_ref[...]
        y = jax.nn.relu(_layer_norm(y, g1_ref[...], bb1_ref[...]))
        zs.append(lax.dot_general(y, wc1_ref[...], (((1,), (1,)), ((), ())),
                                  preferred_element_type=jnp.float32))
    z0_ref[...] = jnp.concatenate(zs[:4], axis=1)
    z1_ref[...] = jnp.concatenate(zs[4:], axis=1)


def _post_body(u0_ref, u1_ref, x_ref, d_ref, wc2_ref, bc1_ref, bc2_ref,
               g2_ref, bb2_ref, w2_ref, b2_ref, o_ref):
    wb = jnp.sum(wc2_ref[...] * bc1_ref[...][None, :], axis=1)
    u0 = u0_ref[...]
    u1 = u1_ref[...]
    db = d_ref[...]
    for i in range(8):
        ui = (u0 if i < 4 else u1)[:, (i % 4) * 16:(i % 4) * 16 + 16]
        c2 = lax.dot_general(ui, wc2_ref[...], (((1,), (1,)), ((), ())),
                             preferred_element_type=jnp.float32)
        c2 = c2 + db * wb[None, :] + bc2_ref[...]
        t = jax.nn.relu(_layer_norm(c2, g2_ref[...], bb2_ref[...]))
        y = lax.dot_general(t, w2_ref[...], (((1,), (1,)), ((), ())),
                            preferred_element_type=jnp.float32) + b2_ref[...]
        o_ref[i] = x_ref[i] + y


def _rep(shape):
    return pl.BlockSpec(shape, lambda nb: (0,) * len(shape))


def kernel(x, incident_matrix, ln_pre_g, ln_pre_b, lin1_W, lin1_b, ln1_g,
           ln1_b, conv1_W, conv1_b, conv2_W, conv2_b, ln2_g, ln2_b, lin2_W,
           lin2_b):
    B, n, C = x.shape
    R = 1024
    grid = (NP // R,)

    Z0, Z1 = pl.pallas_call(
        _pre_body,
        grid=grid,
        in_specs=[
            pl.BlockSpec((B, R, C), lambda nb: (0, nb, 0)),
            _rep((C,)), _rep((C,)),
            _rep((32, C)), _rep((32,)), _rep((32,)), _rep((32,)),
            _rep((16, 32)),
        ],
        out_specs=[pl.BlockSpec((R, W), lambda nb: (nb, 0)),
                   pl.BlockSpec((R, W), lambda nb: (nb, 0))],
        out_shape=[jax.ShapeDtypeStruct((NP, W), jnp.float32),
                   jax.ShapeDtypeStruct((NP, W), jnp.float32)],
    )(x, ln_pre_g, ln_pre_b, lin1_W, lin1_b, ln1_g, ln1_b, conv1_W)

    idx = incident_matrix.astype(jnp.int32).reshape(2, 16, NBLK, BLK)
    U0, U1, d = _sc_prop(Z0, Z1, idx[0], idx[1])

    out = pl.pallas_call(
        _post_body,
        grid=grid,
        in_specs=[
            pl.BlockSpec((R, W), lambda nb: (nb, 0)),
            pl.BlockSpec((R, W), lambda nb: (nb, 0)),
            pl.BlockSpec((B, R, C), lambda nb: (0, nb, 0)),
            pl.BlockSpec((R, 1), lambda nb: (nb, 0)),
            _rep((64, 16)), _rep((16,)), _rep((64,)),
            _rep((64,)), _rep((64,)),
            _rep((C, 64)), _rep((C,)),
        ],
        out_specs=pl.BlockSpec((B, R, C), lambda nb: (0, nb, 0)),
        out_shape=jax.ShapeDtypeStruct((B, n, C), jnp.float32),
    )(U0, U1, x, d[:, None], conv2_W, conv1_b, conv2_b, ln2_g, ln2_b,
      lin2_W, lin2_b)
    return out


# speedup vs baseline: 1.1914x; 1.0618x over previous
import functools

import jax
import jax.numpy as jnp
from jax import lax
from jax.experimental import pallas as pl
from jax.experimental.pallas import tpu as pltpu
from jax.experimental.pallas import tpu_sc as plsc

N = 10000
NP = 10240
SLAB = 640
NNZ = 160000
NBLK = 125
BLK = 80
W = 64

_GDN = lax.GatherDimensionNumbers(
    offset_dims=(), collapsed_slice_dims=(0,), start_index_map=(0,))


SLABC = 128
NCHUNK = SLAB // SLABC


def _sc_body(z0_hbm, z1_hbm, nidx_hbm, eidx_hbm, u0_hbm, u1_hbm, d_hbm,
             bufS, bufT, Dn, Be,
             nidx_v, eidx_v, rows_v, rows2_v, rows3_v, slab_v, binv_v,
             dinv_v, ones_v, gsem0, gsem1, gsem2,
             ssem0, ssem1, ssem2, dsem):
    c = lax.axis_index("c")
    s = lax.axis_index("s")
    r0 = s * SLAB

    zvec = jnp.zeros((16,), jnp.float32)
    onevec = jnp.ones((16,), jnp.float32)

    def fill_slab_zeros():
        def fz(i, _):
            for c4 in range(4):
                slab_v[i, pl.ds(c4 * 16, 16)] = zvec
            return 0
        lax.fori_loop(0, SLABC, fz, 0)

    for j in range(BLK // 16):
        ones_v[pl.ds(j * 16, 16)] = onevec
    for j in range(SLAB // 16):
        binv_v[pl.ds(j * 16, 16)] = zvec

    pltpu.sync_copy(binv_v, Dn.at[pl.ds(r0, SLAB)])
    pltpu.sync_copy(binv_v, Be.at[pl.ds(r0, SLAB)])
    plsc.subcore_barrier()

    pltpu.sync_copy(nidx_hbm.at[s], nidx_v)
    pltpu.sync_copy(eidx_hbm.at[s], eidx_v)

    def deg_body(j, _):
        pltpu.async_copy(ones_v, Dn.at[nidx_v.at[j]], dsem, add=True)
        pltpu.async_copy(ones_v, Be.at[eidx_v.at[j]], dsem, add=True)
        return 0
    lax.fori_loop(0, NBLK, deg_body, 0)

    for k in range(NCHUNK):
        ck = pl.ds(r0 + k * SLABC, SLABC)

        @pl.when(c == 0)
        def _():
            pltpu.sync_copy(z0_hbm.at[ck], slab_v)

        @pl.when(c == 1)
        def _():
            pltpu.sync_copy(z1_hbm.at[ck], slab_v)

        pltpu.sync_copy(slab_v, bufS.at[ck])
    fill_slab_zeros()
    for k in range(NCHUNK):
        pltpu.sync_copy(slab_v, bufT.at[pl.ds(r0 + k * SLABC, SLABC)])

    def deg_drain(j, _):
        pltpu.make_async_copy(ones_v, Dn.at[nidx_v.at[0]], dsem).wait()
        pltpu.make_async_copy(ones_v, Be.at[eidx_v.at[0]], dsem).wait()
        return 0
    lax.fori_loop(0, NBLK, deg_drain, 0)
    plsc.subcore_barrier()

    pltpu.sync_copy(Be.at[pl.ds(r0, SLAB)], binv_v)
    pltpu.sync_copy(Dn.at[pl.ds(r0, SLAB)], dinv_v)

    def inv_body(i, _):
        be = binv_v[pl.ds(i * 16, 16)]
        binv_v[pl.ds(i * 16, 16)] = jnp.where(be > 0, 1.0 / be, 0.0)
        dn = dinv_v[pl.ds(i * 16, 16)]
        dinv_v[pl.ds(i * 16, 16)] = jnp.where(dn > 0, 1.0 / dn, 0.0)
        return 0
    lax.fori_loop(0, SLAB // 16, inv_body, 0)

    @pl.when(c == 0)
    def _():
        pltpu.sync_copy(dinv_v, d_hbm.at[pl.ds(r0, SLAB)])

    def pass_fn(src, dst, sidx, didx):
        rbufs = (rows_v, rows2_v, rows3_v)
        gsems = (gsem0, gsem1, gsem2)
        ssems = (ssem0, ssem1, ssem2)
        pltpu.async_copy(src.at[sidx.at[0]], rbufs[0], gsems[0])
        pltpu.async_copy(src.at[sidx.at[1]], rbufs[1], gsems[1])

        def section(j, b, prefetch):
            pltpu.make_async_copy(src.at[sidx.at[j]], rbufs[b],
                                  gsems[b]).wait()
            pltpu.async_copy(rbufs[b], dst.at[didx.at[j]], ssems[b],
                             add=True)
            if prefetch:
                b2 = (b + 2) % 3

                @pl.when(j >= 1)
                def _():
                    pltpu.make_async_copy(rbufs[b2], dst.at[didx.at[0]],
                                          ssems[b2]).wait()
                pltpu.async_copy(src.at[sidx.at[j + 2]], rbufs[b2],
                                 gsems[b2])

        def super3(i, _):
            for b in range(3):
                section(i * 3 + b, b, True)
            return 0
        lax.fori_loop(0, NBLK // 3, super3, 0)
        for j in range(NBLK - NBLK % 3, NBLK):
            section(j, j % 3, False)
        for j in range(NBLK - 3, NBLK):
            pltpu.make_async_copy(rbufs[j % 3], dst.at[didx.at[0]],
                                  ssems[j % 3]).wait()
        plsc.subcore_barrier()

    def scale_chunk(scalevec, k):
        def sgroup(g, _):
            chunk = scalevec[pl.ds(k * SLABC + g * 16, 16)]
            for i in range(16):
                sv = lax.gather(
                    chunk, jnp.full((16, 1), i, jnp.int32), _GDN, (1,),
                    mode=lax.GatherScatterMode.PROMISE_IN_BOUNDS)
                r = g * 16 + i
                for c4 in range(4):
                    slab_v[r, pl.ds(c4 * 16, 16)] = (
                        slab_v[r, pl.ds(c4 * 16, 16)] * sv)
            return 0
        lax.fori_loop(0, SLABC // 16, sgroup, 0)

    def scale_zero(buf, scalevec, other):
        for k in range(NCHUNK):
            ck = pl.ds(r0 + k * SLABC, SLABC)
            pltpu.sync_copy(buf.at[ck], slab_v)
            scale_chunk(scalevec, k)
            pltpu.sync_copy(slab_v, buf.at[ck])
        fill_slab_zeros()
        for k in range(NCHUNK):
            pltpu.sync_copy(slab_v, other.at[pl.ds(r0 + k * SLABC, SLABC)])
        plsc.subcore_barrier()

    pass_fn(bufS, bufT, nidx_v, eidx_v)
    scale_zero(bufT, binv_v, bufS)
    pass_fn(bufT, bufS, eidx_v, nidx_v)
    scale_zero(bufS, dinv_v, bufT)
    pass_fn(bufS, bufT, nidx_v, eidx_v)
    scale_zero(bufT, binv_v, bufS)
    pass_fn(bufT, bufS, eidx_v, nidx_v)

    @pl.when(c == 0)
    def _():
        pltpu.sync_copy(bufS.at[pl.ds(r0, SLAB)], u0_hbm.at[pl.ds(r0, SLAB)])

    @pl.when(c == 1)
    def _():
        pltpu.sync_copy(bufS.at[pl.ds(r0, SLAB)], u1_hbm.at[pl.ds(r0, SLAB)])


_sc_prop = functools.partial(
    pl.kernel,
    out_type=[jax.ShapeDtypeStruct((NP, W), jnp.float32),
              jax.ShapeDtypeStruct((NP, W), jnp.float32),
              jax.ShapeDtypeStruct((NP,), jnp.float32)],
    mesh=plsc.VectorSubcoreMesh(core_axis_name="c", subcore_axis_name="s"),
    compiler_params=pltpu.CompilerParams(use_tc_tiling_on_sc=False),
    scratch_types=[
        pltpu.VMEM_SHARED((NP, W), jnp.float32),
        pltpu.VMEM_SHARED((NP, W), jnp.float32),
        pltpu.VMEM_SHARED((NP,), jnp.float32),
        pltpu.VMEM_SHARED((NP,), jnp.float32),
        pltpu.VMEM((NBLK, BLK), jnp.int32),
        pltpu.VMEM((NBLK, BLK), jnp.int32),
        pltpu.VMEM((BLK, W), jnp.float32),
        pltpu.VMEM((BLK, W), jnp.float32),
        pltpu.VMEM((BLK, W), jnp.float32),
        pltpu.VMEM((SLABC, W), jnp.float32),
        pltpu.VMEM((SLAB,), jnp.float32),
        pltpu.VMEM((SLAB,), jnp.float32),
        pltpu.VMEM((BLK,), jnp.float32),
        pltpu.SemaphoreType.DMA,
        pltpu.SemaphoreType.DMA,
        pltpu.SemaphoreType.DMA,
        pltpu.SemaphoreType.DMA,
        pltpu.SemaphoreType.DMA,
        pltpu.SemaphoreType.DMA,
        pltpu.SemaphoreType.DMA,
    ],
)(_sc_body)


def _layer_norm(v, g, b):
    on = jnp.full((v.shape[-1], 1), 1.0 / v.shape[-1], jnp.float32)
    mu = lax.dot_general(v, on, (((1,), (0,)), ((), ())),
                         preferred_element_type=jnp.float32)
    m2 = lax.dot_general(v * v, on, (((1,), (0,)), ((), ())),
                         preferred_element_type=jnp.float32)
    var = m2 - mu * mu
    return (v - mu) * lax.rsqrt(var + 1e-5) * g + b


def _pre_body(x_ref, lng_ref, lnb_ref, w1_ref, b1_ref, g1_ref, bb1_ref,
              wc1_ref, z0_ref, z1_ref):
    zs = []
    for i in range(8):
        y = jax.nn.relu(_layer_norm(x_ref[i], lng_ref[...], lnb_ref[...]))
        y = lax.dot_general(y, w1_ref[...], (((1,), (1,)), ((), ())),
                            preferred_element_type=jnp.float32) + b1_ref[...]
        y = jax.nn.relu(_layer_norm(y, g1_ref[...], bb1_ref[...]))
        zs.append(lax.dot_general(y, wc1_ref[...], (((1,), (1,)), ((), ())),
                                  preferred_element_type=jnp.float32))
    z0_ref[...] = jnp.concatenate(zs[:4], axis=1)
    z1_ref[...] = jnp.concatenate(zs[4:], axis=1)


def _post_body(u0_ref, u1_ref, x_ref, d_ref, wc2_ref, bc1_ref, bc2_ref,
               g2_ref, bb2_ref, w2_ref, b2_ref, o_ref):
    wb = jnp.sum(wc2_ref[...] * bc1_ref[...][None, :], axis=1)
    di = d_ref[...]
    u0 = u0_ref[...] * di
    u1 = u1_ref[...] * di
    db = jnp.where(di > 0, 1.0, 0.0)
    for i in range(8):
        ui = (u0 if i < 4 else u1)[:, (i % 4) * 16:(i % 4) * 16 + 16]
        c2 = lax.dot_general(ui, wc2_ref[...], (((1,), (1,)), ((), ())),
                             preferred_element_type=jnp.float32)
        c2 = c2 + db * wb[None, :] + bc2_ref[...]
        t = jax.nn.relu(_layer_norm(c2, g2_ref[...], bb2_ref[...]))
        y = lax.dot_general(t, w2_ref[...], (((1,), (1,)), ((), ())),
                            preferred_element_type=jnp.float32) + b2_ref[...]
        o_ref[i] = x_ref[i] + y


def _rep(shape):
    return pl.BlockSpec(shape, lambda nb: (0,) * len(shape))


def kernel(x, incident_matrix, ln_pre_g, ln_pre_b, lin1_W, lin1_b, ln1_g,
           ln1_b, conv1_W, conv1_b, conv2_W, conv2_b, ln2_g, ln2_b, lin2_W,
           lin2_b):
    B, n, C = x.shape
    R = 2048
    grid = (NP // R,)

    Z0, Z1 = pl.pallas_call(
        _pre_body,
        grid=grid,
        in_specs=[
            pl.BlockSpec((B, R, C), lambda nb: (0, nb, 0)),
            _rep((C,)), _rep((C,)),
            _rep((32, C)), _rep((32,)), _rep((32,)), _rep((32,)),
            _rep((16, 32)),
        ],
        out_specs=[pl.BlockSpec((R, W), lambda nb: (nb, 0)),
                   pl.BlockSpec((R, W), lambda nb: (nb, 0))],
        out_shape=[jax.ShapeDtypeStruct((NP, W), jnp.float32),
                   jax.ShapeDtypeStruct((NP, W), jnp.float32)],
    )(x, ln_pre_g, ln_pre_b, lin1_W, lin1_b, ln1_g, ln1_b, conv1_W)

    idx = incident_matrix.astype(jnp.int32).reshape(2, 16, NBLK, BLK)
    U0, U1, d = _sc_prop(Z0, Z1, idx[0], idx[1])

    out = pl.pallas_call(
        _post_body,
        grid=grid,
        in_specs=[
            pl.BlockSpec((R, W), lambda nb: (nb, 0)),
            pl.BlockSpec((R, W), lambda nb: (nb, 0)),
            pl.BlockSpec((B, R, C), lambda nb: (0, nb, 0)),
            pl.BlockSpec((R, 1), lambda nb: (nb, 0)),
            _rep((64, 16)), _rep((16,)), _rep((64,)),
            _rep((64,)), _rep((64,)),
            _rep((C, 64)), _rep((C,)),
        ],
        out_specs=pl.BlockSpec((B, R, C), lambda nb: (0, nb, 0)),
        out_shape=jax.ShapeDtypeStruct((B, n, C), jnp.float32),
    )(U0, U1, x, d[:, None], conv2_W, conv1_b, conv2_b, ln2_g, ln2_b,
      lin2_W, lin2_b)
    return out


# speedup vs baseline: 1.2097x; 1.0153x over previous
import functools

import jax
import jax.numpy as jnp
from jax import lax
from jax.experimental import pallas as pl
from jax.experimental.pallas import tpu as pltpu
from jax.experimental.pallas import tpu_sc as plsc

N = 10000
NP = 10240
SLAB = 640
NNZ = 160000
NBLK = 125
BLK = 80
W = 64

_GDN = lax.GatherDimensionNumbers(
    offset_dims=(), collapsed_slice_dims=(0,), start_index_map=(0,))


SLABC = 64
NCHUNK = SLAB // SLABC


def _sc_body(z0_hbm, z1_hbm, nidx_hbm, eidx_hbm, u0_hbm, u1_hbm, d_hbm,
             bufS, bufT, Dn, Be,
             nidx_v, eidx_v, rows_v, rows2_v, rows3_v, slab_v, slab2_v,
             binv_v, dinv_v, ones_v, gsem0, gsem1, gsem2,
             ssem0, ssem1, ssem2, dsem, csem0, csem1):
    c = lax.axis_index("c")
    s = lax.axis_index("s")
    r0 = s * SLAB

    zvec = jnp.zeros((16,), jnp.float32)
    onevec = jnp.ones((16,), jnp.float32)

    def fill_slab_zeros():
        def fz(i, _):
            for c4 in range(4):
                slab_v[i, pl.ds(c4 * 16, 16)] = zvec
            return 0
        lax.fori_loop(0, SLABC, fz, 0)

    for j in range(BLK // 16):
        ones_v[pl.ds(j * 16, 16)] = onevec
    for j in range(SLAB // 16):
        binv_v[pl.ds(j * 16, 16)] = zvec

    pltpu.sync_copy(binv_v, Dn.at[pl.ds(r0, SLAB)])
    pltpu.sync_copy(binv_v, Be.at[pl.ds(r0, SLAB)])
    plsc.subcore_barrier()

    pltpu.sync_copy(nidx_hbm.at[s], nidx_v)
    pltpu.sync_copy(eidx_hbm.at[s], eidx_v)

    def deg_body(j, _):
        pltpu.async_copy(ones_v, Dn.at[nidx_v.at[j]], dsem, add=True)
        pltpu.async_copy(ones_v, Be.at[eidx_v.at[j]], dsem, add=True)
        return 0
    lax.fori_loop(0, NBLK, deg_body, 0)

    @pl.when(c == 0)
    def _():
        pltpu.sync_copy(z0_hbm.at[pl.ds(r0, SLAB)], bufS.at[pl.ds(r0, SLAB)])

    @pl.when(c == 1)
    def _():
        pltpu.sync_copy(z1_hbm.at[pl.ds(r0, SLAB)], bufS.at[pl.ds(r0, SLAB)])
    fill_slab_zeros()
    for k in range(NCHUNK):
        pltpu.async_copy(slab_v, bufT.at[pl.ds(r0 + k * SLABC, SLABC)],
                         csem0)
    for k in range(NCHUNK):
        pltpu.make_async_copy(slab_v, bufT.at[pl.ds(r0, SLABC)],
                              csem0).wait()

    def deg_drain(j, _):
        pltpu.make_async_copy(ones_v, Dn.at[nidx_v.at[0]], dsem).wait()
        pltpu.make_async_copy(ones_v, Be.at[eidx_v.at[0]], dsem).wait()
        return 0
    lax.fori_loop(0, NBLK, deg_drain, 0)
    plsc.subcore_barrier()

    pltpu.sync_copy(Be.at[pl.ds(r0, SLAB)], binv_v)
    pltpu.sync_copy(Dn.at[pl.ds(r0, SLAB)], dinv_v)

    def inv_body(i, _):
        be = binv_v[pl.ds(i * 16, 16)]
        binv_v[pl.ds(i * 16, 16)] = jnp.where(be > 0, 1.0 / be, 0.0)
        dn = dinv_v[pl.ds(i * 16, 16)]
        dinv_v[pl.ds(i * 16, 16)] = jnp.where(dn > 0, 1.0 / dn, 0.0)
        return 0
    lax.fori_loop(0, SLAB // 16, inv_body, 0)

    @pl.when(c == 0)
    def _():
        pltpu.sync_copy(dinv_v, d_hbm.at[pl.ds(r0, SLAB)])

    def pass_fn(src, dst, sidx, didx):
        rbufs = (rows_v, rows2_v, rows3_v)
        gsems = (gsem0, gsem1, gsem2)
        ssems = (ssem0, ssem1, ssem2)
        pltpu.async_copy(src.at[sidx.at[0]], rbufs[0], gsems[0])
        pltpu.async_copy(src.at[sidx.at[1]], rbufs[1], gsems[1])

        def section(j, b, prefetch):
            pltpu.make_async_copy(src.at[sidx.at[j]], rbufs[b],
                                  gsems[b]).wait()
            pltpu.async_copy(rbufs[b], dst.at[didx.at[j]], ssems[b],
                             add=True)
            if prefetch:
                b2 = (b + 2) % 3

                @pl.when(j >= 1)
                def _():
                    pltpu.make_async_copy(rbufs[b2], dst.at[didx.at[0]],
                                          ssems[b2]).wait()
                pltpu.async_copy(src.at[sidx.at[j + 2]], rbufs[b2],
                                 gsems[b2])

        def super3(i, _):
            for b in range(3):
                section(i * 3 + b, b, True)
            return 0
        lax.fori_loop(0, NBLK // 3, super3, 0)
        for j in range(NBLK - NBLK % 3, NBLK):
            section(j, j % 3, False)
        for j in range(NBLK - 3, NBLK):
            pltpu.make_async_copy(rbufs[j % 3], dst.at[didx.at[0]],
                                  ssems[j % 3]).wait()
        plsc.subcore_barrier()

    def scale_chunk(scalevec, k, sbuf):
        def sgroup(g, _):
            chunk = scalevec[pl.ds(k * SLABC + g * 16, 16)]
            for i in range(16):
                sv = lax.gather(
                    chunk, jnp.full((16, 1), i, jnp.int32), _GDN, (1,),
                    mode=lax.GatherScatterMode.PROMISE_IN_BOUNDS)
                r = g * 16 + i
                for c4 in range(4):
                    sbuf[r, pl.ds(c4 * 16, 16)] = (
                        sbuf[r, pl.ds(c4 * 16, 16)] * sv)
            return 0
        lax.fori_loop(0, SLABC // 16, sgroup, 0)

    def scale_zero(buf, scalevec, other):
        sbufs = (slab_v, slab2_v)
        csems = (csem0, csem1)
        pltpu.async_copy(buf.at[pl.ds(r0, SLABC)], sbufs[0], csems[0])
        for k in range(NCHUNK):
            b = k % 2
            pltpu.make_async_copy(buf.at[pl.ds(r0, SLABC)], sbufs[b],
                                  csems[b]).wait()
            if k + 1 < NCHUNK:
                ck1 = pl.ds(r0 + (k + 1) * SLABC, SLABC)
                pltpu.async_copy(buf.at[ck1], sbufs[(k + 1) % 2],
                                 csems[(k + 1) % 2])
            scale_chunk(scalevec, k, sbufs[b])
            pltpu.sync_copy(sbufs[b], buf.at[pl.ds(r0 + k * SLABC, SLABC)])
        fill_slab_zeros()
        for k in range(NCHUNK):
            pltpu.async_copy(slab_v, other.at[pl.ds(r0 + k * SLABC, SLABC)],
                             csem0)
        for k in range(NCHUNK):
            pltpu.make_async_copy(slab_v, other.at[pl.ds(r0, SLABC)],
                                  csem0).wait()
        plsc.subcore_barrier()

    pass_fn(bufS, bufT, nidx_v, eidx_v)
    scale_zero(bufT, binv_v, bufS)
    pass_fn(bufT, bufS, eidx_v, nidx_v)
    scale_zero(bufS, dinv_v, bufT)
    pass_fn(bufS, bufT, nidx_v, eidx_v)
    scale_zero(bufT, binv_v, bufS)
    pass_fn(bufT, bufS, eidx_v, nidx_v)

    @pl.when(c == 0)
    def _():
        pltpu.sync_copy(bufS.at[pl.ds(r0, SLAB)], u0_hbm.at[pl.ds(r0, SLAB)])

    @pl.when(c == 1)
    def _():
        pltpu.sync_copy(bufS.at[pl.ds(r0, SLAB)], u1_hbm.at[pl.ds(r0, SLAB)])


_sc_prop = functools.partial(
    pl.kernel,
    out_type=[jax.ShapeDtypeStruct((NP, W), jnp.float32),
              jax.ShapeDtypeStruct((NP, W), jnp.float32),
              jax.ShapeDtypeStruct((NP,), jnp.float32)],
    mesh=plsc.VectorSubcoreMesh(core_axis_name="c", subcore_axis_name="s"),
    compiler_params=pltpu.CompilerParams(use_tc_tiling_on_sc=False),
    scratch_types=[
        pltpu.VMEM_SHARED((NP, W), jnp.float32),
        pltpu.VMEM_SHARED((NP, W), jnp.float32),
        pltpu.VMEM_SHARED((NP,), jnp.float32),
        pltpu.VMEM_SHARED((NP,), jnp.float32),
        pltpu.VMEM((NBLK, BLK), jnp.int32),
        pltpu.VMEM((NBLK, BLK), jnp.int32),
        pltpu.VMEM((BLK, W), jnp.float32),
        pltpu.VMEM((BLK, W), jnp.float32),
        pltpu.VMEM((BLK, W), jnp.float32),
        pltpu.VMEM((SLABC, W), jnp.float32),
        pltpu.VMEM((SLABC, W), jnp.float32),
        pltpu.VMEM((SLAB,), jnp.float32),
        pltpu.VMEM((SLAB,), jnp.float32),
        pltpu.VMEM((BLK,), jnp.float32),
        pltpu.SemaphoreType.DMA,
        pltpu.SemaphoreType.DMA,
        pltpu.SemaphoreType.DMA,
        pltpu.SemaphoreType.DMA,
        pltpu.SemaphoreType.DMA,
        pltpu.SemaphoreType.DMA,
        pltpu.SemaphoreType.DMA,
        pltpu.SemaphoreType.DMA,
        pltpu.SemaphoreType.DMA,
    ],
)(_sc_body)


def _layer_norm(v, g, b):
    on = jnp.full((v.shape[-1], 1), 1.0 / v.shape[-1], jnp.float32)
    mu = lax.dot_general(v, on, (((1,), (0,)), ((), ())),
                         preferred_element_type=jnp.float32)
    m2 = lax.dot_general(v * v, on, (((1,), (0,)), ((), ())),
                         preferred_element_type=jnp.float32)
    var = m2 - mu * mu
    return (v - mu) * lax.rsqrt(var + 1e-5) * g + b


def _pre_body(x_ref, lng_ref, lnb_ref, w1_ref, b1_ref, g1_ref, bb1_ref,
              wc1_ref, z0_ref, z1_ref):
    zs = []
    for i in range(8):
        y = jax.nn.relu(_layer_norm(x_ref[i], lng_ref[...], lnb_ref[...]))
        y = lax.dot_general(y, w1_ref[...], (((1,), (1,)), ((), ())),
                            preferred_element_type=jnp.float32) + b1_ref[...]
        y = jax.nn.relu(_layer_norm(y, g1_ref[...], bb1_ref[...]))
        zs.append(lax.dot_general(y, wc1_ref[...], (((1,), (1,)), ((), ())),
                                  preferred_element_type=jnp.float32))
    z0_ref[...] = jnp.concatenate(zs[:4], axis=1)
    z1_ref[...] = jnp.concatenate(zs[4:], axis=1)


def _post_body(u0_ref, u1_ref, x_ref, d_ref, wc2_ref, bc1_ref, bc2_ref,
               g2_ref, bb2_ref, w2_ref, b2_ref, o_ref):
    wb = jnp.sum(wc2_ref[...] * bc1_ref[...][None, :], axis=1)
    di = d_ref[...]
    u0 = u0_ref[...] * di
    u1 = u1_ref[...] * di
    db = jnp.where(di > 0, 1.0, 0.0)
    for i in range(8):
        ui = (u0 if i < 4 else u1)[:, (i % 4) * 16:(i % 4) * 16 + 16]
        c2 = lax.dot_general(ui, wc2_ref[...], (((1,), (1,)), ((), ())),
                             preferred_element_type=jnp.float32)
        c2 = c2 + db * wb[None, :] + bc2_ref[...]
        t = jax.nn.relu(_layer_norm(c2, g2_ref[...], bb2_ref[...]))
        y = lax.dot_general(t, w2_ref[...], (((1,), (1,)), ((), ())),
                            preferred_element_type=jnp.float32) + b2_ref[...]
        o_ref[i] = x_ref[i] + y


def _rep(shape):
    return pl.BlockSpec(shape, lambda nb: (0,) * len(shape))


def kernel(x, incident_matrix, ln_pre_g, ln_pre_b, lin1_W, lin1_b, ln1_g,
           ln1_b, conv1_W, conv1_b, conv2_W, conv2_b, ln2_g, ln2_b, lin2_W,
           lin2_b):
    B, n, C = x.shape
    R = 2048
    grid = (NP // R,)

    Z0, Z1 = pl.pallas_call(
        _pre_body,
        grid=grid,
        in_specs=[
            pl.BlockSpec((B, R, C), lambda nb: (0, nb, 0)),
            _rep((C,)), _rep((C,)),
            _rep((32, C)), _rep((32,)), _rep((32,)), _rep((32,)),
            _rep((16, 32)),
        ],
        out_specs=[pl.BlockSpec((R, W), lambda nb: (nb, 0)),
                   pl.BlockSpec((R, W), lambda nb: (nb, 0))],
        out_shape=[jax.ShapeDtypeStruct((NP, W), jnp.float32),
                   jax.ShapeDtypeStruct((NP, W), jnp.float32)],
    )(x, ln_pre_g, ln_pre_b, lin1_W, lin1_b, ln1_g, ln1_b, conv1_W)

    idx = incident_matrix.astype(jnp.int32).reshape(2, 16, NBLK, BLK)
    U0, U1, d = _sc_prop(Z0, Z1, idx[0], idx[1])

    out = pl.pallas_call(
        _post_body,
        grid=grid,
        in_specs=[
            pl.BlockSpec((R, W), lambda nb: (nb, 0)),
            pl.BlockSpec((R, W), lambda nb: (nb, 0)),
            pl.BlockSpec((B, R, C), lambda nb: (0, nb, 0)),
            pl.BlockSpec((R, 1), lambda nb: (nb, 0)),
            _rep((64, 16)), _rep((16,)), _rep((64,)),
            _rep((64,)), _rep((64,)),
            _rep((C, 64)), _rep((C,)),
        ],
        out_specs=pl.BlockSpec((B, R, C), lambda nb: (0, nb, 0)),
        out_shape=jax.ShapeDtypeStruct((B, n, C), jnp.float32),
    )(U0, U1, x, d[:, None], conv2_W, conv1_b, conv2_b, ln2_g, ln2_b,
      lin2_W, lin2_b)
    return out
